# baseline (speedup 1.0000x reference)
import functools

import jax
import jax.numpy as jnp
from jax import lax
from jax.experimental import pallas as pl
from jax.experimental.pallas import tpu as pltpu
from jax.experimental.pallas import tpu_sc as plsc

NS = 16
LANES = 16

L = 16384
D = 64
NTAGS = 16
VOCAB = 1_000_000
CPAD = 1_048_576
PER_T = L // NS
CHUNK = CPAD // NS

BLK = 49152
NBLK = (VOCAB + BLK - 1) // BLK
SUBS = BLK // 128
TAIL_SUBS = (VOCAB - (NBLK - 1) * BLK + 127) // 128


def _count_sc(words3):
    mesh = plsc.VectorSubcoreMesh(
        core_axis_name="c", subcore_axis_name="s", num_cores=1
    )

    @functools.partial(
        pl.kernel,
        out_type=jax.ShapeDtypeStruct((CPAD,), jnp.float32),
        mesh=mesh,
        scratch_types=[
            pltpu.VMEM((PER_T // 128, 128), jnp.int32),
            pltpu.VMEM((CHUNK // 4,), jnp.float32),
            pltpu.VMEM((128,), jnp.float32),
            pltpu.VMEM_SHARED((CPAD,), jnp.float32),
        ],
    )
    def k(words_hbm, cnt_hbm, idx_v, zeros_v, ones_v, cnt_s):
        sid = lax.axis_index("s")

        z16 = jnp.zeros((LANES,), jnp.float32)
        o16 = jnp.ones((LANES,), jnp.float32)

        def zbody(i, _):
            zeros_v[pl.ds(i * LANES, LANES)] = z16
            return 0

        lax.fori_loop(0, (CHUNK // 4) // LANES, zbody, 0)
        for i in range(128 // LANES):
            ones_v[pl.ds(i * LANES, LANES)] = o16

        pltpu.sync_copy(words_hbm.at[sid], idx_v)
        for q in range(4):
            pltpu.sync_copy(
                zeros_v, cnt_s.at[pl.ds(sid * CHUNK + q * (CHUNK // 4),
                                        CHUNK // 4)]
            )
        plsc.subcore_barrier()

        for j in range(PER_T // 128):
            pltpu.sync_copy(ones_v, cnt_s.at[idx_v.at[j]], add=True)
        plsc.subcore_barrier()

        pltpu.sync_copy(
            cnt_s.at[pl.ds(sid * CHUNK, CHUNK)],
            cnt_hbm.at[pl.ds(sid * CHUNK, CHUNK)],
        )

    return k(words3)


def _matvec_tc(table_t, cnt2, wt, b2):
    def body(tbl_ref, cnt_ref, wt_ref, b_ref, out_ref, acc_ref, row_ref):
        j = pl.program_id(0)

        @pl.when(j == 0)
        def _():
            acc_ref[...] = jnp.zeros((D, 128), jnp.float32)

        def accumulate(nsubs, mask_tail):
            accs = [acc_ref[v * 16:(v + 1) * 16, :] for v in range(D // 16)]
            for sub in range(nsubs):
                row = cnt_ref[sub:sub + 1, :]
                rowb = jnp.broadcast_to(row, (16, 128))
                if mask_tail:
                    base = (NBLK - 1) * BLK + sub * 128
                    ok = base + lax.iota(jnp.int32, 128) < VOCAB
                    rowb = jnp.where(ok[None, :], rowb, 0.0)
                row_ref[...] = rowb
                rowm = row_ref[...]
                for v in range(D // 16):
                    t = tbl_ref[v * 16:(v + 1) * 16, sub * 128:(sub + 1) * 128]
                    if mask_tail:
                        t = jnp.where(ok[None, :], t, 0.0)
                    accs[v] = accs[v] + t * rowm
            for v in range(D // 16):
                acc_ref[v * 16:(v + 1) * 16, :] = accs[v]

        @pl.when(j < NBLK - 1)
        def _():
            accumulate(SUBS, False)

        @pl.when(j == NBLK - 1)
        def _():
            accumulate(TAIL_SUBS, True)
            emb = jnp.sum(acc_ref[...], axis=1)
            logits = jnp.sum(wt_ref[...] * emb[:, None], axis=0)
            out_ref[...] = logits[None, :] + b_ref[...]

        return

    return pl.pallas_call(
        body,
        grid=(NBLK,),
        in_specs=[
            pl.BlockSpec((D, BLK), lambda j: (0, j)),
            pl.BlockSpec((SUBS, 128), lambda j: (j, 0)),
            pl.BlockSpec((D, NTAGS), lambda j: (0, 0)),
            pl.BlockSpec((1, NTAGS), lambda j: (0, 0)),
        ],
        out_specs=pl.BlockSpec((1, NTAGS), lambda j: (0, 0)),
        out_shape=jax.ShapeDtypeStruct((1, NTAGS), jnp.float32),
        scratch_shapes=[
            pltpu.VMEM((D, 128), jnp.float32),
            pltpu.VMEM((16, 128), jnp.float32),
        ],
        compiler_params=pltpu.CompilerParams(
            dimension_semantics=("arbitrary",),
        ),
    )(table_t, cnt2, wt, b2)


def kernel(words, embedding, W, b):
    words3 = words.astype(jnp.int32).reshape(NS, PER_T // 128, 128)
    cnt = _count_sc(words3)
    cnt2 = cnt.reshape(CPAD // 128, 128)
    table_t = embedding.T
    wt = W.T
    b2 = b.reshape(1, NTAGS)
    return _matvec_tc(table_t, cnt2, wt, b2)


# speedup vs baseline: 1.0093x; 1.0093x over previous
import functools

import jax
import jax.numpy as jnp
from jax import lax
from jax.experimental import pallas as pl
from jax.experimental.pallas import tpu as pltpu
from jax.experimental.pallas import tpu_sc as plsc

NS = 16
LANES = 16

L = 16384
D = 64
NTAGS = 16
VOCAB = 1_000_000
CPAD = 1_048_576
PER_T = L // NS
CHUNK = CPAD // NS

BLK = 32768
NBLK = (VOCAB + BLK - 1) // BLK
SUBS = BLK // 128
TAIL_SUBS = (VOCAB - (NBLK - 1) * BLK + 127) // 128


def _count_sc(words3):
    mesh = plsc.VectorSubcoreMesh(
        core_axis_name="c", subcore_axis_name="s", num_cores=1
    )

    @functools.partial(
        pl.kernel,
        out_type=jax.ShapeDtypeStruct((CPAD,), jnp.float32),
        mesh=mesh,
        scratch_types=[
            pltpu.VMEM((PER_T // 128, 128), jnp.int32),
            pltpu.VMEM((CHUNK // 4,), jnp.float32),
            pltpu.VMEM((128,), jnp.float32),
            pltpu.VMEM_SHARED((CPAD,), jnp.float32),
        ],
    )
    def k(words_hbm, cnt_hbm, idx_v, zeros_v, ones_v, cnt_s):
        sid = lax.axis_index("s")

        z16 = jnp.zeros((LANES,), jnp.float32)
        o16 = jnp.ones((LANES,), jnp.float32)

        def zbody(i, _):
            zeros_v[pl.ds(i * LANES, LANES)] = z16
            return 0

        lax.fori_loop(0, (CHUNK // 4) // LANES, zbody, 0)
        for i in range(128 // LANES):
            ones_v[pl.ds(i * LANES, LANES)] = o16

        pltpu.sync_copy(words_hbm.at[sid], idx_v)
        for q in range(4):
            pltpu.sync_copy(
                zeros_v, cnt_s.at[pl.ds(sid * CHUNK + q * (CHUNK // 4),
                                        CHUNK // 4)]
            )
        plsc.subcore_barrier()

        for j in range(PER_T // 128):
            pltpu.sync_copy(ones_v, cnt_s.at[idx_v.at[j]], add=True)
        plsc.subcore_barrier()

        pltpu.sync_copy(
            cnt_s.at[pl.ds(sid * CHUNK, CHUNK)],
            cnt_hbm.at[pl.ds(sid * CHUNK, CHUNK)],
        )

    return k(words3)


def _matvec_tc(table_t, cnt2, wt, b2):
    def body(tbl_ref, cnt_ref, wt_ref, b_ref, out_ref, acc_ref, row_ref):
        j = pl.program_id(0)

        @pl.when(j == 0)
        def _():
            acc_ref[...] = jnp.zeros((D, 128), jnp.float32)

        def accumulate(nsubs, mask_tail):
            accs = [acc_ref[v * 16:(v + 1) * 16, :] for v in range(D // 16)]
            for sub in range(nsubs):
                row = cnt_ref[sub:sub + 1, :]
                rowb = jnp.broadcast_to(row, (16, 128))
                if mask_tail:
                    base = (NBLK - 1) * BLK + sub * 128
                    ok = base + lax.iota(jnp.int32, 128) < VOCAB
                    rowb = jnp.where(ok[None, :], rowb, 0.0)
                row_ref[...] = rowb
                rowm = row_ref[...]
                for v in range(D // 16):
                    t = tbl_ref[v * 16:(v + 1) * 16, sub * 128:(sub + 1) * 128]
                    if mask_tail:
                        t = jnp.where(ok[None, :], t, 0.0)
                    accs[v] = accs[v] + t * rowm
            for v in range(D // 16):
                acc_ref[v * 16:(v + 1) * 16, :] = accs[v]

        @pl.when(j < NBLK - 1)
        def _():
            accumulate(SUBS, False)

        @pl.when(j == NBLK - 1)
        def _():
            accumulate(TAIL_SUBS, True)
            emb = jnp.sum(acc_ref[...], axis=1)
            logits = jnp.sum(wt_ref[...] * emb[:, None], axis=0)
            out_ref[...] = logits[None, :] + b_ref[...]

        return

    return pl.pallas_call(
        body,
        grid=(NBLK,),
        in_specs=[
            pl.BlockSpec((D, BLK), lambda j: (0, j)),
            pl.BlockSpec((SUBS, 128), lambda j: (j, 0)),
            pl.BlockSpec((D, NTAGS), lambda j: (0, 0)),
            pl.BlockSpec((1, NTAGS), lambda j: (0, 0)),
        ],
        out_specs=pl.BlockSpec((1, NTAGS), lambda j: (0, 0)),
        out_shape=jax.ShapeDtypeStruct((1, NTAGS), jnp.float32),
        scratch_shapes=[
            pltpu.VMEM((D, 128), jnp.float32),
            pltpu.VMEM((16, 128), jnp.float32),
        ],
        compiler_params=pltpu.CompilerParams(
            dimension_semantics=("arbitrary",),
        ),
    )(table_t, cnt2, wt, b2)


def kernel(words, embedding, W, b):
    words3 = words.astype(jnp.int32).reshape(NS, PER_T // 128, 128)
    cnt = _count_sc(words3)
    cnt2 = cnt.reshape(CPAD // 128, 128)
    table_t = embedding.T
    wt = W.T
    b2 = b.reshape(1, NTAGS)
    return _matvec_tc(table_t, cnt2, wt, b2)


# speedup vs baseline: 1.0213x; 1.0119x over previous
import functools

import jax
import jax.numpy as jnp
from jax import lax
from jax.experimental import pallas as pl
from jax.experimental.pallas import tpu as pltpu
from jax.experimental.pallas import tpu_sc as plsc

NS = 16
LANES = 16

L = 16384
D = 64
NTAGS = 16
VOCAB = 1_000_000
CPAD = 1_048_576
PER_T = L // NS
CHUNK = CPAD // NS

BLK = 32768
NBLK = (VOCAB + BLK - 1) // BLK
SUBS = BLK // 128
TAIL_SUBS = (VOCAB - (NBLK - 1) * BLK + 127) // 128


def _count_sc(words3):
    mesh = plsc.VectorSubcoreMesh(
        core_axis_name="c", subcore_axis_name="s", num_cores=1
    )

    @functools.partial(
        pl.kernel,
        out_type=jax.ShapeDtypeStruct((CPAD,), jnp.float32),
        mesh=mesh,
        scratch_types=[
            pltpu.VMEM((PER_T // 128, 128), jnp.int32),
            pltpu.VMEM((CHUNK // 4,), jnp.float32),
            pltpu.VMEM((128,), jnp.float32),
            pltpu.VMEM_SHARED((CPAD,), jnp.float32),
        ],
    )
    def k(words_hbm, cnt_hbm, idx_v, zeros_v, ones_v, cnt_s):
        sid = lax.axis_index("s")

        z16 = jnp.zeros((LANES,), jnp.float32)
        o16 = jnp.ones((LANES,), jnp.float32)

        def zbody(i, _):
            zeros_v[pl.ds(i * LANES, LANES)] = z16
            return 0

        lax.fori_loop(0, (CHUNK // 4) // LANES, zbody, 0)
        for i in range(128 // LANES):
            ones_v[pl.ds(i * LANES, LANES)] = o16

        pltpu.sync_copy(words_hbm.at[sid], idx_v)
        for q in range(4):
            pltpu.sync_copy(
                zeros_v, cnt_s.at[pl.ds(sid * CHUNK + q * (CHUNK // 4),
                                        CHUNK // 4)]
            )
        plsc.subcore_barrier()

        for j in range(PER_T // 128):
            pltpu.sync_copy(ones_v, cnt_s.at[idx_v.at[j]], add=True)
        plsc.subcore_barrier()

        pltpu.sync_copy(
            cnt_s.at[pl.ds(sid * CHUNK, CHUNK)],
            cnt_hbm.at[pl.ds(sid * CHUNK, CHUNK)],
        )

    return k(words3)


def _matvec_tc(table_t, cnt2, wt, b2):
    def body(tbl_ref, cnt_ref, wt_ref, b_ref, out_ref, acc_ref):
        j = pl.program_id(0)

        @pl.when(j == 0)
        def _():
            acc_ref[...] = jnp.zeros((D, 128), jnp.float32)

        def accumulate(nsubs, mask_tail):
            acc = acc_ref[...]
            for sub in range(nsubs):
                t = tbl_ref[:, sub * 128:(sub + 1) * 128]
                if mask_tail:
                    base = (NBLK - 1) * BLK + sub * 128
                    ok = base + lax.iota(jnp.int32, 128) < VOCAB
                    t = jnp.where(ok[None, :], t, 0.0)
                row = cnt_ref[sub * 128:(sub + 1) * 128]
                acc = acc + t * row[None, :]
            acc_ref[...] = acc

        @pl.when(j < NBLK - 1)
        def _():
            accumulate(SUBS, False)

        @pl.when(j == NBLK - 1)
        def _():
            accumulate(TAIL_SUBS, True)
            emb = jnp.sum(acc_ref[...], axis=1)
            logits = jnp.sum(wt_ref[...] * emb[:, None], axis=0)
            out_ref[...] = logits[None, :] + b_ref[...]

        return

    return pl.pallas_call(
        body,
        grid=(NBLK,),
        in_specs=[
            pl.BlockSpec((D, BLK), lambda j: (0, j)),
            pl.BlockSpec((BLK,), lambda j: (j,)),
            pl.BlockSpec((D, NTAGS), lambda j: (0, 0)),
            pl.BlockSpec((1, NTAGS), lambda j: (0, 0)),
        ],
        out_specs=pl.BlockSpec((1, NTAGS), lambda j: (0, 0)),
        out_shape=jax.ShapeDtypeStruct((1, NTAGS), jnp.float32),
        scratch_shapes=[pltpu.VMEM((D, 128), jnp.float32)],
        compiler_params=pltpu.CompilerParams(
            dimension_semantics=("arbitrary",),
        ),
    )(table_t, cnt2, wt, b2)


def kernel(words, embedding, W, b):
    words3 = words.astype(jnp.int32).reshape(NS, PER_T // 128, 128)
    cnt = _count_sc(words3)
    table_t = embedding.T
    wt = W.T
    b2 = b.reshape(1, NTAGS)
    return _matvec_tc(table_t, cnt, wt, b2)
